# Initial kernel scaffold; baseline (speedup 1.0000x reference)
import jax
import jax.numpy as jnp
from jax.experimental import pallas as pl


def kernel(node_emb, edge_embeddings, b_s, b_r, bases_s, coeff_s, bases_r, coeff_r, edge_index, edge_type):
    raise NotImplementedError("write your pallas kernel here")


# speedup vs baseline: 112.1546x; 112.1546x over previous
import functools

import jax
import jax.numpy as jnp
from jax import lax
from jax.experimental import pallas as pl
from jax.experimental.pallas import tpu as pltpu
from jax.experimental.pallas import tpu_sc as plsc

_N = 10000
_E = 320000
_D = 128
_RD = 16
_R = 32
_NB = 16
_IN_S = 2 * _D + _RD
_NR = _N * _R

_NUM_CORES = 2
_NUM_SUBCORES = 16
_NUM_WORKERS = _NUM_CORES * _NUM_SUBCORES
_EPT = _E // _NUM_WORKERS
_ROW_W = 128
_ROWS = (_EPT + _ROW_W - 1) // _ROW_W
_EPT_PAD = _ROWS * _ROW_W
_BPT = _NR // _NUM_SUBCORES


def _sc_count_body(edge_dst_hbm, edge_type_hbm, out_hbm,
                   dst_v, typ_v, idx_v, ones_v, stage_v, hist_sh):
    cid = lax.axis_index("c")
    sid = lax.axis_index("s")
    wid = sid * _NUM_CORES + cid
    base = wid * _EPT

    zvec = jnp.zeros((16,), jnp.float32)

    def _zero(i, carry):
        stage_v[pl.ds(i * 16, 16)] = zvec
        return carry

    lax.fori_loop(0, _BPT // 16, _zero, 0)
    pltpu.sync_copy(stage_v, hist_sh.at[pl.ds(sid * _BPT, _BPT)])

    pltpu.sync_copy(edge_dst_hbm.at[pl.ds(base, _EPT)],
                    dst_v.at[pl.ds(0, _EPT)])
    pltpu.sync_copy(edge_type_hbm.at[pl.ds(base, _EPT)],
                    typ_v.at[pl.ds(0, _EPT)])

    for i in range(_ROW_W // 16):
        ones_v[pl.ds(i * 16, 16)] = jnp.ones((16,), jnp.float32)

    def _col(k, j):
        off = j * _ROW_W + k * 16
        d = dst_v[pl.ds(off, 16)]
        t = typ_v[pl.ds(off, 16)]
        idx_v[j, pl.ds(k * 16, 16)] = d * _R + t
        return k + 1

    def _row(j, carry):
        lax.fori_loop(0, _ROW_W // 16, lambda k, _: _col(k, j) * 0, 0)
        return carry

    lax.fori_loop(0, _ROWS, _row, 0)

    pad_vec = jnp.full((16,), _NR, jnp.int32)
    for i in range((_EPT_PAD - _EPT) // 16):
        idx_v[_ROWS - 1, pl.ds(_EPT - (_ROWS - 1) * _ROW_W + i * 16, 16)] = pad_vec

    plsc.subcore_barrier()

    def _scat(j, carry):
        pltpu.sync_copy(ones_v, hist_sh.at[idx_v.at[j]], add=True)
        return carry

    lax.fori_loop(0, _ROWS, _scat, 0)

    plsc.subcore_barrier()

    pltpu.sync_copy(hist_sh.at[pl.ds(sid * _BPT, _BPT)], stage_v)
    pltpu.sync_copy(stage_v, out_hbm.at[pl.ds(cid * _NR + sid * _BPT, _BPT)])


_sc_count = pl.kernel(
    _sc_count_body,
    out_type=jax.ShapeDtypeStruct((_NUM_CORES * _NR,), jnp.float32),
    mesh=plsc.VectorSubcoreMesh(core_axis_name="c", subcore_axis_name="s"),
    scratch_types=[
        pltpu.VMEM((_EPT_PAD,), jnp.int32),
        pltpu.VMEM((_EPT_PAD,), jnp.int32),
        pltpu.VMEM((_ROWS, _ROW_W), jnp.int32),
        pltpu.VMEM((_ROW_W,), jnp.float32),
        pltpu.VMEM((_BPT,), jnp.float32),
        pltpu.VMEM_SHARED((_NR + 16,), jnp.float32),
    ],
)


_BLK = 1000


def _tc_attn_body(x_ref, cnt0_ref, cnt1_ref, eemb_ref, br_ref, basr_ref,
                  coefr_ref, out_ref):
    x = x_ref[...]
    rw = jnp.dot(coefr_ref[...], basr_ref[...],
                 preferred_element_type=jnp.float32)
    wp = rw[:, :_D]
    wq = rw[:, _D:2 * _D]
    wr = rw[:, 2 * _D:]
    p = lax.dot_general(x, wp, (((1,), (1,)), ((), ())),
                        preferred_element_type=jnp.float32)
    q = lax.dot_general(x, wq, (((1,), (1,)), ((), ())),
                        preferred_element_type=jnp.float32)
    dvec = jnp.sum(eemb_ref[...] * wr, axis=1)[None, :] + br_ref[...]
    mask = (cnt0_ref[...] + cnt1_ref[...]) > 0.0
    score = p + jnp.where(mask, q, 0.0) + dvec
    score = jnp.where(mask, score, jnp.float32(-10000000000.0))
    m = jnp.max(score, axis=1, keepdims=True)
    e = jnp.exp(score - m)
    z = jnp.sum(e, axis=1, keepdims=True)
    s = jnp.sum(jnp.where(mask, e, 0.0), axis=1, keepdims=True) / z
    out_ref[...] = x * (1.0 + s)


_tc_attn = pl.pallas_call(
    _tc_attn_body,
    grid=(_N // _BLK,),
    in_specs=[
        pl.BlockSpec((_BLK, _D), lambda i: (i, 0)),
        pl.BlockSpec((_BLK, _R), lambda i: (i, 0)),
        pl.BlockSpec((_BLK, _R), lambda i: (i, 0)),
        pl.BlockSpec((_R, _RD), lambda i: (0, 0)),
        pl.BlockSpec((1, _R), lambda i: (0, 0)),
        pl.BlockSpec((_NB, _IN_S), lambda i: (0, 0)),
        pl.BlockSpec((_R, _NB), lambda i: (0, 0)),
    ],
    out_specs=pl.BlockSpec((_BLK, _D), lambda i: (i, 0)),
    out_shape=jax.ShapeDtypeStruct((_N, _D), jnp.float32),
)


def kernel(node_emb, edge_embeddings, b_s, b_r, bases_s, coeff_s, bases_r,
           coeff_r, edge_index, edge_type):
    del b_s, bases_s, coeff_s
    counts = _sc_count(edge_index[1], edge_type)
    counts = counts.reshape(_NUM_CORES, _N, _R)
    return _tc_attn(node_emb, counts[0], counts[1], edge_embeddings,
                    b_r.reshape(1, _R), bases_r[:, :, 0], coeff_r)


# speedup vs baseline: 150.2505x; 1.3397x over previous
import functools

import jax
import jax.numpy as jnp
from jax import lax
from jax.experimental import pallas as pl
from jax.experimental.pallas import tpu as pltpu
from jax.experimental.pallas import tpu_sc as plsc

_N = 10000
_E = 320000
_D = 128
_RD = 16
_R = 32
_NB = 16
_IN_S = 2 * _D + _RD
_RPAD = 128
_NBINS = _N * _RPAD

_NUM_CORES = 2
_NUM_SUBCORES = 16
_NUM_WORKERS = _NUM_CORES * _NUM_SUBCORES
_ROW_W = 128
_ROWS = 79
_EPT_PAD = _ROWS * _ROW_W
_BASE_ROWS = 78
_EXTRA_FROM = _NUM_WORKERS - 4
_BPT = _NBINS // _NUM_SUBCORES
_CH = 4000
_NCH = _BPT // _CH
_DEPTH = 8


def _sc_count_body(edge_index_hbm, edge_type_hbm, out_hbm,
                   ei_v, typ_v, idx_v, ones_v, stage_a, stage_b, hist_sh,
                   sem, sem2, sem_r, sem_w):
    cid = lax.axis_index("c")
    sid = lax.axis_index("s")
    wid = sid * _NUM_CORES + cid
    base = (wid * _BASE_ROWS + jnp.maximum(wid - _EXTRA_FROM, 0)) * _ROW_W

    in1 = pltpu.make_async_copy(
        edge_index_hbm.at[:, pl.ds(base, _EPT_PAD)], ei_v, sem2)
    in1.start()
    in2 = pltpu.make_async_copy(
        edge_type_hbm.at[pl.ds(base, _EPT_PAD)], typ_v, sem2)
    in2.start()

    zvec = jnp.zeros((16,), jnp.float32)

    def _zero(i, carry):
        stage_a[pl.ds(i * 16, 16)] = zvec
        return carry

    lax.fori_loop(0, _CH // 16, _zero, 0)

    def _zcopy(q, carry):
        pltpu.sync_copy(stage_a,
                        hist_sh.at[pl.ds(sid * _BPT + q * _CH, _CH)])
        return carry

    lax.fori_loop(0, _NCH, _zcopy, 0)

    for i in range(_ROW_W // 16):
        ones_v[pl.ds(i * 16, 16)] = jnp.ones((16,), jnp.float32)

    in1.wait()
    in2.wait()

    def _col(k, j):
        off = j * _ROW_W + k * 16
        d = ei_v[1, pl.ds(off, 16)]
        t = typ_v[pl.ds(off, 16)]
        idx_v[j, pl.ds(k * 16, 16)] = d * _RPAD + t
        return k + 1

    def _row(j, carry):
        lax.fori_loop(0, _ROW_W // 16, lambda k, _: _col(k, j) * 0, 0)
        return carry

    lax.fori_loop(0, _ROWS, _row, 0)

    pad_vec = jnp.full((16,), _NBINS, jnp.int32)

    @pl.when(wid < _EXTRA_FROM)
    def _():
        for i in range(_ROW_W // 16):
            idx_v[_ROWS - 1, pl.ds(i * 16, 16)] = pad_vec

    plsc.subcore_barrier()

    def _fire(j):
        pltpu.make_async_copy(ones_v, hist_sh.at[idx_v.at[j]], sem).start()

    def _wait_one():
        pltpu.make_async_copy(ones_v, hist_sh.at[idx_v.at[0]], sem).wait()

    for j in range(_DEPTH):
        _fire(j)

    def _pipe(j, carry):
        _fire(j)
        _wait_one()
        return carry

    lax.fori_loop(_DEPTH, _ROWS, _pipe, 0)
    for _ in range(_DEPTH):
        _wait_one()

    plsc.subcore_barrier()

    bufs = (stage_a, stage_b)

    def _rd(r):
        return pltpu.make_async_copy(
            hist_sh.at[pl.ds(sid * _BPT + r * _CH, _CH)],
            bufs[r % 2], sem_r)

    def _wr(r):
        return pltpu.make_async_copy(
            bufs[r % 2],
            out_hbm.at[pl.ds(cid * _NBINS + sid * _BPT + r * _CH, _CH)],
            sem_w)

    _rd(0).start()
    for r in range(_NCH):
        _rd(r).wait()
        _wr(r).start()
        if r + 1 < _NCH:
            if r >= 1:
                _wr(r - 1).wait()
            _rd(r + 1).start()
    _wr(_NCH - 2).wait()
    _wr(_NCH - 1).wait()


_sc_count = pl.kernel(
    _sc_count_body,
    out_type=jax.ShapeDtypeStruct((_NUM_CORES * _NBINS,), jnp.float32),
    mesh=plsc.VectorSubcoreMesh(core_axis_name="c", subcore_axis_name="s"),
    scratch_types=[
        pltpu.VMEM((2, _EPT_PAD), jnp.int32),
        pltpu.VMEM((_EPT_PAD,), jnp.int32),
        pltpu.VMEM((_ROWS, _ROW_W), jnp.int32),
        pltpu.VMEM((_ROW_W,), jnp.float32),
        pltpu.VMEM((_CH,), jnp.float32),
        pltpu.VMEM((_CH,), jnp.float32),
        pltpu.VMEM_SHARED((_NBINS + 16,), jnp.float32),
        pltpu.SemaphoreType.DMA,
        pltpu.SemaphoreType.DMA,
        pltpu.SemaphoreType.DMA,
        pltpu.SemaphoreType.DMA,
    ],
)


_BLK = 1000


def _tc_attn_body(x_ref, cnt0_ref, cnt1_ref, eemb_ref, br_ref, basr_ref,
                  coefr_ref, out_ref):
    x = x_ref[...]
    basr = basr_ref[...][:, :, 0]
    rw = jnp.dot(coefr_ref[...], basr,
                 preferred_element_type=jnp.float32)
    wp = rw[:, :_D]
    wq = rw[:, _D:2 * _D]
    wr = rw[:, 2 * _D:]
    p = lax.dot_general(x, wp, (((1,), (1,)), ((), ())),
                        preferred_element_type=jnp.float32)
    q = lax.dot_general(x, wq, (((1,), (1,)), ((), ())),
                        preferred_element_type=jnp.float32)
    dvec = jnp.sum(eemb_ref[...] * wr, axis=1) + br_ref[...][:, 0]
    cnt = cnt0_ref[...][0, :, :_R] + cnt1_ref[...][0, :, :_R]
    mask = cnt > 0.0
    score = p + jnp.where(mask, q, 0.0) + dvec[None, :]
    score = jnp.where(mask, score, jnp.float32(-10000000000.0))
    m = jnp.max(score, axis=1, keepdims=True)
    e = jnp.exp(score - m)
    z = jnp.sum(e, axis=1, keepdims=True)
    s = jnp.sum(jnp.where(mask, e, 0.0), axis=1, keepdims=True) / z
    out_ref[...] = x * (1.0 + s)


_tc_attn = pl.pallas_call(
    _tc_attn_body,
    grid=(_N // _BLK,),
    in_specs=[
        pl.BlockSpec((_BLK, _D), lambda i: (i, 0)),
        pl.BlockSpec((1, _BLK, _RPAD), lambda i: (0, i, 0)),
        pl.BlockSpec((1, _BLK, _RPAD), lambda i: (1, i, 0)),
        pl.BlockSpec((_R, _RD), lambda i: (0, 0)),
        pl.BlockSpec((_R, 1), lambda i: (0, 0)),
        pl.BlockSpec((_NB, _IN_S, 1), lambda i: (0, 0, 0)),
        pl.BlockSpec((_R, _NB), lambda i: (0, 0)),
    ],
    out_specs=pl.BlockSpec((_BLK, _D), lambda i: (i, 0)),
    out_shape=jax.ShapeDtypeStruct((_N, _D), jnp.float32),
)


def kernel(node_emb, edge_embeddings, b_s, b_r, bases_s, coeff_s, bases_r,
           coeff_r, edge_index, edge_type):
    del b_s, bases_s, coeff_s
    counts = _sc_count(edge_index, edge_type)
    counts3 = counts.reshape(_NUM_CORES, _N, _RPAD)
    return _tc_attn(node_emb, counts3, counts3, edge_embeddings, b_r,
                    bases_r, coeff_r)


# speedup vs baseline: 156.6792x; 1.0428x over previous
import functools

import jax
import jax.numpy as jnp
from jax import lax
from jax.experimental import pallas as pl
from jax.experimental.pallas import tpu as pltpu
from jax.experimental.pallas import tpu_sc as plsc

_N = 10000
_E = 320000
_D = 128
_RD = 16
_R = 32
_NB = 16
_IN_S = 2 * _D + _RD
_RPAD = 128
_NBINS = _N * _RPAD

_NUM_CORES = 2
_NUM_SUBCORES = 16
_NUM_WORKERS = _NUM_CORES * _NUM_SUBCORES
_ROW_W = 128
_ROWS = 79
_EPT_PAD = _ROWS * _ROW_W
_BASE_ROWS = 78
_EXTRA_FROM = _NUM_WORKERS - 4
_BPT = _NBINS // _NUM_SUBCORES
_CH = 4000
_NCH = _BPT // _CH
_DEPTH = 16


def _sc_count_body(edge_index_hbm, edge_type_hbm, out_hbm,
                   ei_v, typ_v, idx_v, ones_v, stage_a, stage_b, hist_sh,
                   sem, sem2, sem_r, sem_w):
    cid = lax.axis_index("c")
    sid = lax.axis_index("s")
    wid = sid * _NUM_CORES + cid
    base = (wid * _BASE_ROWS + jnp.maximum(wid - _EXTRA_FROM, 0)) * _ROW_W

    in1 = pltpu.make_async_copy(
        edge_index_hbm.at[:, pl.ds(base, _EPT_PAD)], ei_v, sem2)
    in1.start()
    in2 = pltpu.make_async_copy(
        edge_type_hbm.at[pl.ds(base, _EPT_PAD)], typ_v, sem2)
    in2.start()

    zvec = jnp.zeros((16,), jnp.float32)

    def _zero(i, carry):
        stage_a[pl.ds(i * 16, 16)] = zvec
        return carry

    lax.fori_loop(0, _CH // 16, _zero, 0)

    def _zcopy(q, carry):
        pltpu.make_async_copy(
            stage_a, hist_sh.at[pl.ds(sid * _BPT + q * _CH, _CH)],
            sem_r).start()
        return carry

    lax.fori_loop(0, _NCH, _zcopy, 0)

    for i in range(_ROW_W // 16):
        ones_v[pl.ds(i * 16, 16)] = jnp.ones((16,), jnp.float32)

    in1.wait()
    in2.wait()

    def _row(j, carry):
        for k in range(_ROW_W // 16):
            off = j * _ROW_W + k * 16
            d = ei_v[1, pl.ds(off, 16)]
            t = typ_v[pl.ds(off, 16)]
            idx_v[j, pl.ds(k * 16, 16)] = d * _RPAD + t
        return carry

    lax.fori_loop(0, _ROWS, _row, 0)

    pad_vec = jnp.full((16,), _NBINS, jnp.int32)

    @pl.when(wid < _EXTRA_FROM)
    def _():
        for i in range(_ROW_W // 16):
            idx_v[_ROWS - 1, pl.ds(i * 16, 16)] = pad_vec

    def _zdrain(q, carry):
        pltpu.make_async_copy(
            stage_a, hist_sh.at[pl.ds(sid * _BPT, _CH)], sem_r).wait()
        return carry

    lax.fori_loop(0, _NCH, _zdrain, 0)

    plsc.subcore_barrier()

    def _fire(j):
        pltpu.make_async_copy(ones_v, hist_sh.at[idx_v.at[j]], sem).start()

    def _wait_one():
        pltpu.make_async_copy(ones_v, hist_sh.at[idx_v.at[0]], sem).wait()

    for j in range(_DEPTH):
        _fire(j)

    def _pipe(j, carry):
        _fire(j)
        _wait_one()
        return carry

    lax.fori_loop(_DEPTH, _ROWS, _pipe, 0)
    for _ in range(_DEPTH):
        _wait_one()

    plsc.subcore_barrier()

    def _rd(c, buf):
        return pltpu.make_async_copy(
            hist_sh.at[pl.ds(sid * _BPT + c * _CH, _CH)], buf, sem_r)

    def _wr(c, buf):
        return pltpu.make_async_copy(
            buf, out_hbm.at[pl.ds(cid * _NBINS + sid * _BPT + c * _CH, _CH)],
            sem_w)

    _rd(0, stage_a).start()

    def _cp(r, carry):
        c0 = 2 * r
        _rd(c0, stage_a).wait()
        _rd(c0 + 1, stage_b).start()
        _wr(c0, stage_a).start()
        _rd(c0 + 1, stage_b).wait()
        _wr(c0 + 1, stage_b).start()
        _wr(c0, stage_a).wait()

        @pl.when(r < _NCH // 2 - 1)
        def _():
            _rd(c0 + 2, stage_a).start()

        _wr(c0 + 1, stage_b).wait()
        return carry

    lax.fori_loop(0, _NCH // 2, _cp, 0)


_sc_count = pl.kernel(
    _sc_count_body,
    out_type=jax.ShapeDtypeStruct((_NUM_CORES * _NBINS,), jnp.float32),
    mesh=plsc.VectorSubcoreMesh(core_axis_name="c", subcore_axis_name="s"),
    scratch_types=[
        pltpu.VMEM((2, _EPT_PAD), jnp.int32),
        pltpu.VMEM((_EPT_PAD,), jnp.int32),
        pltpu.VMEM((_ROWS, _ROW_W), jnp.int32),
        pltpu.VMEM((_ROW_W,), jnp.float32),
        pltpu.VMEM((_CH,), jnp.float32),
        pltpu.VMEM((_CH,), jnp.float32),
        pltpu.VMEM_SHARED((_NBINS + 16,), jnp.float32),
        pltpu.SemaphoreType.DMA,
        pltpu.SemaphoreType.DMA,
        pltpu.SemaphoreType.DMA,
        pltpu.SemaphoreType.DMA,
    ],
)


_BLK = 1000


def _tc_attn_body(x_ref, cnt0_ref, cnt1_ref, eemb_ref, br_ref, basr_ref,
                  coefr_ref, out_ref):
    x = x_ref[...]
    basr = basr_ref[...][:, :, 0]
    rw = jnp.dot(coefr_ref[...], basr,
                 preferred_element_type=jnp.float32)
    wp = rw[:, :_D]
    wq = rw[:, _D:2 * _D]
    wr = rw[:, 2 * _D:]
    p = lax.dot_general(x, wp, (((1,), (1,)), ((), ())),
                        preferred_element_type=jnp.float32)
    q = lax.dot_general(x, wq, (((1,), (1,)), ((), ())),
                        preferred_element_type=jnp.float32)
    dvec = jnp.sum(eemb_ref[...] * wr, axis=1) + br_ref[...][:, 0]
    cnt = cnt0_ref[...][0, :, :_R] + cnt1_ref[...][0, :, :_R]
    mask = cnt > 0.0
    score = p + jnp.where(mask, q, 0.0) + dvec[None, :]
    score = jnp.where(mask, score, jnp.float32(-10000000000.0))
    m = jnp.max(score, axis=1, keepdims=True)
    e = jnp.exp(score - m)
    z = jnp.sum(e, axis=1, keepdims=True)
    s = jnp.sum(jnp.where(mask, e, 0.0), axis=1, keepdims=True) / z
    out_ref[...] = x * (1.0 + s)


_tc_attn = pl.pallas_call(
    _tc_attn_body,
    grid=(_N // _BLK,),
    in_specs=[
        pl.BlockSpec((_BLK, _D), lambda i: (i, 0)),
        pl.BlockSpec((1, _BLK, _RPAD), lambda i: (0, i, 0)),
        pl.BlockSpec((1, _BLK, _RPAD), lambda i: (1, i, 0)),
        pl.BlockSpec((_R, _RD), lambda i: (0, 0)),
        pl.BlockSpec((_R, 1), lambda i: (0, 0)),
        pl.BlockSpec((_NB, _IN_S, 1), lambda i: (0, 0, 0)),
        pl.BlockSpec((_R, _NB), lambda i: (0, 0)),
    ],
    out_specs=pl.BlockSpec((_BLK, _D), lambda i: (i, 0)),
    out_shape=jax.ShapeDtypeStruct((_N, _D), jnp.float32),
)


def kernel(node_emb, edge_embeddings, b_s, b_r, bases_s, coeff_s, bases_r,
           coeff_r, edge_index, edge_type):
    del b_s, bases_s, coeff_s
    counts = _sc_count(edge_index, edge_type)
    counts3 = counts.reshape(_NUM_CORES, _N, _RPAD)
    return _tc_attn(node_emb, counts3, counts3, edge_embeddings, b_r,
                    bases_r, coeff_r)


# speedup vs baseline: 167.4299x; 1.0686x over previous
import functools

import jax
import jax.numpy as jnp
from jax import lax
from jax.experimental import pallas as pl
from jax.experimental.pallas import tpu as pltpu
from jax.experimental.pallas import tpu_sc as plsc

_N = 10000
_E = 320000
_D = 128
_RD = 16
_R = 32
_NB = 16
_IN_S = 2 * _D + _RD
_RPAD = 128
_NBINS = _N * _RPAD

_NUM_CORES = 2
_NUM_SUBCORES = 16
_NUM_WORKERS = _NUM_CORES * _NUM_SUBCORES
_ROW_W = 128
_ROWS = 79
_EPT_PAD = _ROWS * _ROW_W
_BASE_ROWS = 78
_EXTRA_FROM = _NUM_WORKERS - 4
_BPT = _NBINS // _NUM_SUBCORES
_CH = 4000
_NCH = _BPT // _CH
_DEPTH = 8


def _sc_count_body(edge_index_hbm, edge_type_hbm, out_hbm,
                   ei_v, typ_v, idx_v, ones_v, stage_a, stage_b, hist_sh,
                   sem, sem2, sem_r, sem_w):
    cid = lax.axis_index("c")
    sid = lax.axis_index("s")
    wid = sid * _NUM_CORES + cid
    base = (wid * _BASE_ROWS + jnp.maximum(wid - _EXTRA_FROM, 0)) * _ROW_W

    in1 = pltpu.make_async_copy(
        edge_index_hbm.at[:, pl.ds(base, _EPT_PAD)], ei_v, sem2)
    in1.start()
    in2 = pltpu.make_async_copy(
        edge_type_hbm.at[pl.ds(base, _EPT_PAD)], typ_v, sem2)
    in2.start()

    zvec = jnp.zeros((16,), jnp.float32)

    def _zero(i, carry):
        stage_a[pl.ds(i * 16, 16)] = zvec
        return carry

    lax.fori_loop(0, _CH // 16, _zero, 0)

    def _zcopy(q, carry):
        pltpu.make_async_copy(
            stage_a, hist_sh.at[pl.ds(sid * _BPT + q * _CH, _CH)],
            sem_r).start()
        return carry

    lax.fori_loop(0, _NCH, _zcopy, 0)

    ovec = jnp.ones((16,), jnp.float32)

    def _ones(i, carry):
        ones_v[pl.ds(i * 16, 16)] = ovec
        return carry

    lax.fori_loop(0, _ROW_W // 16, _ones, 0)

    in1.wait()
    in2.wait()

    def _row(j, carry):
        for k in range(_ROW_W // 16):
            off = j * _ROW_W + k * 16
            d = ei_v[1, pl.ds(off, 16)]
            t = typ_v[pl.ds(off, 16)]
            idx_v[j, pl.ds(k * 16, 16)] = d * _RPAD + t
        return carry

    lax.fori_loop(0, _ROWS, _row, 0)

    pad_vec = jnp.full((16,), _NBINS, jnp.int32)

    @pl.when(wid < _EXTRA_FROM)
    def _():
        def _pad(i, carry):
            idx_v[_ROWS - 1, pl.ds(i * 16, 16)] = pad_vec
            return carry

        lax.fori_loop(0, _ROW_W // 16, _pad, 0)

    def _zdrain(q, carry):
        pltpu.make_async_copy(
            stage_a, hist_sh.at[pl.ds(sid * _BPT, _CH)], sem_r).wait()
        return carry

    lax.fori_loop(0, _NCH, _zdrain, 0)

    plsc.subcore_barrier()

    def _fire(j):
        pltpu.make_async_copy(ones_v, hist_sh.at[idx_v.at[j]], sem).start()

    def _wait_one():
        pltpu.make_async_copy(ones_v, hist_sh.at[idx_v.at[0]], sem).wait()

    for j in range(_DEPTH):
        _fire(j)

    def _pipe(j, carry):
        _fire(j)
        _wait_one()
        return carry

    lax.fori_loop(_DEPTH, _ROWS, _pipe, 0)
    for _ in range(_DEPTH):
        _wait_one()

    plsc.subcore_barrier()

    def _rd(c, buf):
        return pltpu.make_async_copy(
            hist_sh.at[pl.ds(sid * _BPT + c * _CH, _CH)], buf, sem_r)

    def _wr(c, buf):
        return pltpu.make_async_copy(
            buf, out_hbm.at[pl.ds(cid * _NBINS + sid * _BPT + c * _CH, _CH)],
            sem_w)

    _rd(0, stage_a).start()

    def _cp(r, carry):
        c0 = 2 * r
        _rd(c0, stage_a).wait()
        _rd(c0 + 1, stage_b).start()
        _wr(c0, stage_a).start()
        _rd(c0 + 1, stage_b).wait()
        _wr(c0 + 1, stage_b).start()
        _wr(c0, stage_a).wait()

        @pl.when(r < _NCH // 2 - 1)
        def _():
            _rd(c0 + 2, stage_a).start()

        _wr(c0 + 1, stage_b).wait()
        return carry

    lax.fori_loop(0, _NCH // 2, _cp, 0)


_sc_count = pl.kernel(
    _sc_count_body,
    out_type=jax.ShapeDtypeStruct((_NUM_CORES * _NBINS,), jnp.float32),
    mesh=plsc.VectorSubcoreMesh(core_axis_name="c", subcore_axis_name="s"),
    scratch_types=[
        pltpu.VMEM((2, _EPT_PAD), jnp.int32),
        pltpu.VMEM((_EPT_PAD,), jnp.int32),
        pltpu.VMEM((_ROWS, _ROW_W), jnp.int32),
        pltpu.VMEM((_ROW_W,), jnp.float32),
        pltpu.VMEM((_CH,), jnp.float32),
        pltpu.VMEM((_CH,), jnp.float32),
        pltpu.VMEM_SHARED((_NBINS + 16,), jnp.float32),
        pltpu.SemaphoreType.DMA,
        pltpu.SemaphoreType.DMA,
        pltpu.SemaphoreType.DMA,
        pltpu.SemaphoreType.DMA,
    ],
)


_BLK = 2000


def _tc_attn_body(x_ref, cnt0_ref, cnt1_ref, eemb_ref, br_ref, basr_ref,
                  coefr_ref, out_ref):
    x = x_ref[...]
    basr = basr_ref[...][:, :, 0]
    rw = jnp.dot(coefr_ref[...], basr,
                 preferred_element_type=jnp.float32)
    wp = rw[:, :_D]
    wq = rw[:, _D:2 * _D]
    wr = rw[:, 2 * _D:]
    p = lax.dot_general(x, wp, (((1,), (1,)), ((), ())),
                        preferred_element_type=jnp.float32)
    q = lax.dot_general(x, wq, (((1,), (1,)), ((), ())),
                        preferred_element_type=jnp.float32)
    dvec = jnp.sum(eemb_ref[...] * wr, axis=1) + br_ref[...][:, 0]
    cnt = cnt0_ref[...][0, :, :_R] + cnt1_ref[...][0, :, :_R]
    mask = cnt > 0.0
    score = p + jnp.where(mask, q, 0.0) + dvec[None, :]
    score = jnp.where(mask, score, jnp.float32(-10000000000.0))
    m = jnp.max(score, axis=1, keepdims=True)
    e = jnp.exp(score - m)
    z = jnp.sum(e, axis=1, keepdims=True)
    s = jnp.sum(jnp.where(mask, e, 0.0), axis=1, keepdims=True) / z
    out_ref[...] = x * (1.0 + s)


_tc_attn = pl.pallas_call(
    _tc_attn_body,
    grid=(_N // _BLK,),
    in_specs=[
        pl.BlockSpec((_BLK, _D), lambda i: (i, 0)),
        pl.BlockSpec((1, _BLK, _RPAD), lambda i: (0, i, 0)),
        pl.BlockSpec((1, _BLK, _RPAD), lambda i: (1, i, 0)),
        pl.BlockSpec((_R, _RD), lambda i: (0, 0)),
        pl.BlockSpec((_R, 1), lambda i: (0, 0)),
        pl.BlockSpec((_NB, _IN_S, 1), lambda i: (0, 0, 0)),
        pl.BlockSpec((_R, _NB), lambda i: (0, 0)),
    ],
    out_specs=pl.BlockSpec((_BLK, _D), lambda i: (i, 0)),
    out_shape=jax.ShapeDtypeStruct((_N, _D), jnp.float32),
)


def kernel(node_emb, edge_embeddings, b_s, b_r, bases_s, coeff_s, bases_r,
           coeff_r, edge_index, edge_type):
    del b_s, bases_s, coeff_s
    counts = _sc_count(edge_index, edge_type)
    counts3 = counts.reshape(_NUM_CORES, _N, _RPAD)
    return _tc_attn(node_emb, counts3, counts3, edge_embeddings, b_r,
                    bases_r, coeff_r)


# speedup vs baseline: 176.0741x; 1.0516x over previous
import functools

import jax
import jax.numpy as jnp
from jax import lax
from jax.experimental import pallas as pl
from jax.experimental.pallas import tpu as pltpu
from jax.experimental.pallas import tpu_sc as plsc

_N = 10000
_E = 320000
_D = 128
_RD = 16
_R = 32
_NB = 16
_IN_S = 2 * _D + _RD
_FOLD = 4
_NF = 2560
_NBINS = _NF * 128

_NUM_CORES = 2
_NUM_SUBCORES = 16
_NUM_WORKERS = _NUM_CORES * _NUM_SUBCORES
_ROW_W = 128
_ROWS = 79
_EPT_PAD = _ROWS * _ROW_W
_BASE_ROWS = 78
_EXTRA_FROM = _NUM_WORKERS - 4
_BPT = _NBINS // _NUM_SUBCORES
_CH = 10240
_DEPTH = 8


def _sc_count_body(edge_index_hbm, edge_type_hbm, out_hbm,
                   ei_v, typ_v, idx_v, ones_v, stage_a, stage_b, hist_sh,
                   sem, sem2, sem_r, sem_w):
    cid = lax.axis_index("c")
    sid = lax.axis_index("s")
    wid = sid * _NUM_CORES + cid
    base = (wid * _BASE_ROWS + jnp.maximum(wid - _EXTRA_FROM, 0)) * _ROW_W

    in1 = pltpu.make_async_copy(
        edge_index_hbm.at[:, pl.ds(base, _EPT_PAD)], ei_v, sem2)
    in1.start()
    in2 = pltpu.make_async_copy(
        edge_type_hbm.at[pl.ds(base, _EPT_PAD)], typ_v, sem2)
    in2.start()

    zvec = jnp.zeros((16,), jnp.float32)

    def _zero(i, carry):
        stage_a[pl.ds(i * 16, 16)] = zvec
        return carry

    lax.fori_loop(0, _CH // 16, _zero, 0)
    for q in range(_BPT // _CH):
        pltpu.make_async_copy(
            stage_a, hist_sh.at[pl.ds(sid * _BPT + q * _CH, _CH)],
            sem_r).start()

    ovec = jnp.ones((16,), jnp.float32)

    def _ones(i, carry):
        ones_v[pl.ds(i * 16, 16)] = ovec
        return carry

    lax.fori_loop(0, _ROW_W // 16, _ones, 0)

    in1.wait()
    in2.wait()

    def _row(j, carry):
        for k in range(_ROW_W // 16):
            off = j * _ROW_W + k * 16
            d = ei_v[1, pl.ds(off, 16)]
            t = typ_v[pl.ds(off, 16)]
            slot = lax.shift_right_logical(d * 13108, 25)
            idx_v[j, pl.ds(k * 16, 16)] = (
                d * 128 + t - slot * (128 * _NF - _R))
        return carry

    lax.fori_loop(0, _ROWS, _row, 0)

    pad_vec = jnp.full((16,), _NBINS, jnp.int32)

    @pl.when(wid < _EXTRA_FROM)
    def _():
        def _pad(i, carry):
            idx_v[_ROWS - 1, pl.ds(i * 16, 16)] = pad_vec
            return carry

        lax.fori_loop(0, _ROW_W // 16, _pad, 0)

    for q in range(_BPT // _CH):
        pltpu.make_async_copy(
            stage_a, hist_sh.at[pl.ds(sid * _BPT, _CH)], sem_r).wait()

    plsc.subcore_barrier()

    def _fire(j):
        pltpu.make_async_copy(ones_v, hist_sh.at[idx_v.at[j]], sem).start()

    def _wait_one():
        pltpu.make_async_copy(ones_v, hist_sh.at[idx_v.at[0]], sem).wait()

    for j in range(_DEPTH):
        _fire(j)

    def _pipe(j, carry):
        _fire(j)
        _wait_one()
        return carry

    lax.fori_loop(_DEPTH, _ROWS, _pipe, 0)
    for _ in range(_DEPTH):
        _wait_one()

    plsc.subcore_barrier()

    def _rd(c, buf, s):
        return pltpu.make_async_copy(
            hist_sh.at[pl.ds(sid * _BPT + c * _CH, _CH)], buf, s)

    def _wr(c, buf, s):
        return pltpu.make_async_copy(
            buf, out_hbm.at[pl.ds(cid * _NBINS + sid * _BPT + c * _CH, _CH)],
            s)

    _rd(0, stage_a, sem_r).start()
    _rd(1, stage_b, sem_w).start()
    _rd(0, stage_a, sem_r).wait()
    _wr(0, stage_a, sem_r).start()
    _rd(1, stage_b, sem_w).wait()
    _wr(1, stage_b, sem_w).start()
    _wr(0, stage_a, sem_r).wait()
    _wr(1, stage_b, sem_w).wait()


_sc_count = pl.kernel(
    _sc_count_body,
    out_type=jax.ShapeDtypeStruct((_NUM_CORES * _NBINS,), jnp.float32),
    mesh=plsc.VectorSubcoreMesh(core_axis_name="c", subcore_axis_name="s"),
    scratch_types=[
        pltpu.VMEM((2, _EPT_PAD), jnp.int32),
        pltpu.VMEM((_EPT_PAD,), jnp.int32),
        pltpu.VMEM((_ROWS, _ROW_W), jnp.int32),
        pltpu.VMEM((_ROW_W,), jnp.float32),
        pltpu.VMEM((_CH,), jnp.float32),
        pltpu.VMEM((_CH,), jnp.float32),
        pltpu.VMEM_SHARED((_NBINS + 16,), jnp.float32),
        pltpu.SemaphoreType.DMA,
        pltpu.SemaphoreType.DMA,
        pltpu.SemaphoreType.DMA,
        pltpu.SemaphoreType.DMA,
    ],
)


_BLKF = 640


def _tc_attn_body(x_ref, cnt0_ref, cnt1_ref, eemb_ref, br_ref, basr_ref,
                  coefr_ref, out_ref):
    k = pl.program_id(1)
    x = x_ref[...]
    basr = basr_ref[...][:, :, 0]
    rw = jnp.dot(coefr_ref[...], basr,
                 preferred_element_type=jnp.float32)
    wp = rw[:, :_D]
    wq = rw[:, _D:2 * _D]
    wr = rw[:, 2 * _D:]
    p = lax.dot_general(x, wp, (((1,), (1,)), ((), ())),
                        preferred_element_type=jnp.float32)
    q = lax.dot_general(x, wq, (((1,), (1,)), ((), ())),
                        preferred_element_type=jnp.float32)
    dvec = jnp.sum(eemb_ref[...] * wr, axis=1) + br_ref[...][:, 0]
    lane = lax.broadcasted_iota(jnp.int32, (128, _R), 0)
    col = lax.broadcasted_iota(jnp.int32, (128, _R), 1)
    sel = (lane == col + _R * k).astype(jnp.float32)
    csum = cnt0_ref[...][0] + cnt1_ref[...][0]
    cnt = lax.dot_general(csum, sel, (((1,), (0,)), ((), ())),
                          preferred_element_type=jnp.float32)
    mask = cnt > 0.0
    score = p + jnp.where(mask, q, 0.0) + dvec[None, :]
    score = jnp.where(mask, score, jnp.float32(-10000000000.0))
    m = jnp.max(score, axis=1, keepdims=True)
    e = jnp.exp(score - m)
    z = jnp.sum(e, axis=1, keepdims=True)
    s = jnp.sum(jnp.where(mask, e, 0.0), axis=1, keepdims=True) / z
    out_ref[...] = x * (1.0 + s)


_tc_attn = pl.pallas_call(
    _tc_attn_body,
    grid=(_NF // _BLKF, _FOLD),
    in_specs=[
        pl.BlockSpec((_BLKF, _D), lambda i, k: (k * (_NF // _BLKF) + i, 0)),
        pl.BlockSpec((1, _BLKF, 128), lambda i, k: (0, i, 0)),
        pl.BlockSpec((1, _BLKF, 128), lambda i, k: (1, i, 0)),
        pl.BlockSpec((_R, _RD), lambda i, k: (0, 0)),
        pl.BlockSpec((_R, 1), lambda i, k: (0, 0)),
        pl.BlockSpec((_NB, _IN_S, 1), lambda i, k: (0, 0, 0)),
        pl.BlockSpec((_R, _NB), lambda i, k: (0, 0)),
    ],
    out_specs=pl.BlockSpec((_BLKF, _D), lambda i, k: (k * (_NF // _BLKF) + i, 0)),
    out_shape=jax.ShapeDtypeStruct((_N, _D), jnp.float32),
)


def kernel(node_emb, edge_embeddings, b_s, b_r, bases_s, coeff_s, bases_r,
           coeff_r, edge_index, edge_type):
    del b_s, bases_s, coeff_s
    counts = _sc_count(edge_index, edge_type)
    counts3 = counts.reshape(_NUM_CORES, _NF, 128)
    return _tc_attn(node_emb, counts3, counts3, edge_embeddings, b_r,
                    bases_r, coeff_r)


# speedup vs baseline: 207.4402x; 1.1781x over previous
import functools

import jax
import jax.numpy as jnp
from jax import lax
from jax.experimental import pallas as pl
from jax.experimental.pallas import tpu as pltpu
from jax.experimental.pallas import tpu_sc as plsc

_N = 10000
_E = 320000
_D = 128
_RD = 16
_R = 32
_NB = 16
_IN_S = 2 * _D + _RD
_FOLD = 4
_NF = 2560
_NBINS = _NF * 128

_NUM_CORES = 2
_NUM_SUBCORES = 16
_NUM_WORKERS = _NUM_CORES * _NUM_SUBCORES
_ROW_W = 128
_ROWS = 79
_EPT_PAD = _ROWS * _ROW_W
_BASE_ROWS = 78
_EXTRA_FROM = _NUM_WORKERS - 4
_BPT = _NBINS // _NUM_SUBCORES
_CH = 10240
_DEPTH = 8


def _sc_count_body(edge_index_hbm, edge_type_hbm, out_hbm,
                   ei_v, typ_v, idx_v, ones_v, stage_a, stage_b, hist_sh,
                   sem, sem2, sem_r, sem_w):
    cid = lax.axis_index("c")
    sid = lax.axis_index("s")
    wid = sid * _NUM_CORES + cid
    base = (wid * _BASE_ROWS + jnp.maximum(wid - _EXTRA_FROM, 0)) * _ROW_W

    in1 = pltpu.make_async_copy(
        edge_index_hbm.at[:, pl.ds(base, _EPT_PAD)], ei_v, sem2)
    in1.start()
    in2 = pltpu.make_async_copy(
        edge_type_hbm.at[pl.ds(base, _EPT_PAD)], typ_v, sem2)
    in2.start()

    zvec = jnp.zeros((16,), jnp.float32)

    def _zero(i, carry):
        stage_a[pl.ds(i * 16, 16)] = zvec
        return carry

    lax.fori_loop(0, _CH // 16, _zero, 0)
    for q in range(_BPT // _CH):
        pltpu.make_async_copy(
            stage_a, hist_sh.at[pl.ds(sid * _BPT + q * _CH, _CH)],
            sem_r).start()

    ovec = jnp.ones((16,), jnp.float32)

    def _ones(i, carry):
        ones_v[pl.ds(i * 16, 16)] = ovec
        return carry

    lax.fori_loop(0, _ROW_W // 16, _ones, 0)

    in1.wait()
    in2.wait()

    def _row(j, carry):
        for k in range(_ROW_W // 16):
            off = j * _ROW_W + k * 16
            d = ei_v[1, pl.ds(off, 16)]
            t = typ_v[pl.ds(off, 16)]
            slot = lax.shift_right_logical(d * 13108, 25)
            idx_v[j, pl.ds(k * 16, 16)] = (
                d * 128 + t - slot * (128 * _NF - _R))
        return carry

    lax.fori_loop(0, _ROWS, _row, 0)

    pad_vec = jnp.full((16,), _NBINS, jnp.int32)

    @pl.when(wid < _EXTRA_FROM)
    def _():
        def _pad(i, carry):
            idx_v[_ROWS - 1, pl.ds(i * 16, 16)] = pad_vec
            return carry

        lax.fori_loop(0, _ROW_W // 16, _pad, 0)

    for q in range(_BPT // _CH):
        pltpu.make_async_copy(
            stage_a, hist_sh.at[pl.ds(sid * _BPT, _CH)], sem_r).wait()

    plsc.subcore_barrier()

    def _fire(j):
        pltpu.make_async_copy(ones_v, hist_sh.at[idx_v.at[j]], sem).start()

    def _wait_one():
        pltpu.make_async_copy(ones_v, hist_sh.at[idx_v.at[0]], sem).wait()

    for j in range(_DEPTH):
        _fire(j)

    def _pipe(j, carry):
        _fire(j)
        _wait_one()
        return carry

    lax.fori_loop(_DEPTH, _ROWS, _pipe, 0)
    for _ in range(_DEPTH):
        _wait_one()

    plsc.subcore_barrier()

    def _rd(c, buf, s):
        return pltpu.make_async_copy(
            hist_sh.at[pl.ds(sid * _BPT + c * _CH, _CH)], buf, s)

    def _wr(c, buf, s):
        return pltpu.make_async_copy(
            buf, out_hbm.at[pl.ds(cid * _NBINS + sid * _BPT + c * _CH, _CH)],
            s)

    _rd(0, stage_a, sem_r).start()
    _rd(1, stage_b, sem_w).start()
    _rd(0, stage_a, sem_r).wait()
    _wr(0, stage_a, sem_r).start()
    _rd(1, stage_b, sem_w).wait()
    _wr(1, stage_b, sem_w).start()
    _wr(0, stage_a, sem_r).wait()
    _wr(1, stage_b, sem_w).wait()


_sc_count = pl.kernel(
    _sc_count_body,
    out_type=jax.ShapeDtypeStruct((_NUM_CORES * _NBINS,), jnp.float32),
    mesh=plsc.VectorSubcoreMesh(core_axis_name="c", subcore_axis_name="s"),
    scratch_types=[
        pltpu.VMEM((2, _EPT_PAD), jnp.int32),
        pltpu.VMEM((_EPT_PAD,), jnp.int32),
        pltpu.VMEM((_ROWS, _ROW_W), jnp.int32),
        pltpu.VMEM((_ROW_W,), jnp.float32),
        pltpu.VMEM((_CH,), jnp.float32),
        pltpu.VMEM((_CH,), jnp.float32),
        pltpu.VMEM_SHARED((_NBINS + 16,), jnp.float32),
        pltpu.SemaphoreType.DMA,
        pltpu.SemaphoreType.DMA,
        pltpu.SemaphoreType.DMA,
        pltpu.SemaphoreType.DMA,
    ],
)


_BLKF = 2560


def _tc_attn_body(x_ref, cnt0_ref, cnt1_ref, eemb_ref, br_ref, basr_ref,
                  coefr_ref, out_ref):
    k = pl.program_id(1)
    x = x_ref[...]
    basr = basr_ref[...][:, :, 0]
    rw = jnp.dot(coefr_ref[...], basr,
                 preferred_element_type=jnp.float32)
    wp = rw[:, :_D]
    wq = rw[:, _D:2 * _D]
    wr = rw[:, 2 * _D:]
    p = lax.dot_general(x, wp, (((1,), (1,)), ((), ())),
                        preferred_element_type=jnp.float32)
    q = lax.dot_general(x, wq, (((1,), (1,)), ((), ())),
                        preferred_element_type=jnp.float32)
    dvec = jnp.sum(eemb_ref[...] * wr, axis=1) + br_ref[...][:, 0]
    lane = lax.broadcasted_iota(jnp.int32, (128, _R), 0)
    col = lax.broadcasted_iota(jnp.int32, (128, _R), 1)
    sel = (lane == col + _R * k).astype(jnp.float32)
    csum = cnt0_ref[...][0] + cnt1_ref[...][0]
    cnt = lax.dot_general(csum, sel, (((1,), (0,)), ((), ())),
                          preferred_element_type=jnp.float32)
    mask = cnt > 0.0
    score = p + jnp.where(mask, q, 0.0) + dvec[None, :]
    score = jnp.where(mask, score, jnp.float32(-10000000000.0))
    m = jnp.max(score, axis=1, keepdims=True)
    e = jnp.exp(score - m)
    z = jnp.sum(e, axis=1, keepdims=True)
    s = jnp.sum(jnp.where(mask, e, 0.0), axis=1, keepdims=True) / z
    out_ref[...] = x * (1.0 + s)


_tc_attn = pl.pallas_call(
    _tc_attn_body,
    grid=(_NF // _BLKF, _FOLD),
    in_specs=[
        pl.BlockSpec((_BLKF, _D), lambda i, k: (k * (_NF // _BLKF) + i, 0)),
        pl.BlockSpec((1, _BLKF, 128), lambda i, k: (0, i, 0)),
        pl.BlockSpec((1, _BLKF, 128), lambda i, k: (1, i, 0)),
        pl.BlockSpec((_R, _RD), lambda i, k: (0, 0)),
        pl.BlockSpec((_R, 1), lambda i, k: (0, 0)),
        pl.BlockSpec((_NB, _IN_S, 1), lambda i, k: (0, 0, 0)),
        pl.BlockSpec((_R, _NB), lambda i, k: (0, 0)),
    ],
    out_specs=pl.BlockSpec((_BLKF, _D), lambda i, k: (k * (_NF // _BLKF) + i, 0)),
    out_shape=jax.ShapeDtypeStruct((_N, _D), jnp.float32),
)


def kernel(node_emb, edge_embeddings, b_s, b_r, bases_s, coeff_s, bases_r,
           coeff_r, edge_index, edge_type):
    del b_s, bases_s, coeff_s
    counts = _sc_count(edge_index, edge_type)
    counts3 = counts.reshape(_NUM_CORES, _NF, 128)
    return _tc_attn(node_emb, counts3, counts3, edge_embeddings, b_r,
                    bases_r, coeff_r)


# speedup vs baseline: 207.8857x; 1.0021x over previous
import functools

import jax
import jax.numpy as jnp
from jax import lax
from jax.experimental import pallas as pl
from jax.experimental.pallas import tpu as pltpu
from jax.experimental.pallas import tpu_sc as plsc

_N = 10000
_E = 320000
_D = 128
_RD = 16
_R = 32
_NB = 16
_IN_S = 2 * _D + _RD
_FOLD = 4
_NF = 2560
_NBINS = _NF * 128

_NUM_CORES = 2
_NUM_SUBCORES = 16
_NUM_WORKERS = _NUM_CORES * _NUM_SUBCORES
_ROW_W = 128
_ROWS = 79
_EPT_PAD = _ROWS * _ROW_W
_BASE_ROWS = 78
_EXTRA_FROM = _NUM_WORKERS - 4
_BPT = _NBINS // _NUM_SUBCORES
_CH = 10240
_DEPTH = 8


def _sc_count_body(edge_index_hbm, edge_type_hbm, out_hbm,
                   ei_v, typ_v, idx_v, ones_v, stage_a, stage_b, hist_sh,
                   sem, sem2, sem_r, sem_w):
    cid = lax.axis_index("c")
    sid = lax.axis_index("s")
    wid = sid * _NUM_CORES + cid
    base = (wid * _BASE_ROWS + jnp.maximum(wid - _EXTRA_FROM, 0)) * _ROW_W

    in1 = pltpu.make_async_copy(
        edge_index_hbm.at[:, pl.ds(base, _EPT_PAD)], ei_v, sem2)
    in1.start()
    in2 = pltpu.make_async_copy(
        edge_type_hbm.at[pl.ds(base, _EPT_PAD)], typ_v, sem2)
    in2.start()

    zvec = jnp.zeros((16,), jnp.float32)

    def _zero(i, carry):
        stage_a[pl.ds(i * 16, 16)] = zvec
        return carry

    lax.fori_loop(0, _CH // 16, _zero, 0)
    for q in range(_BPT // _CH):
        pltpu.make_async_copy(
            stage_a, hist_sh.at[pl.ds(sid * _BPT + q * _CH, _CH)],
            sem_r).start()

    ovec = jnp.ones((16,), jnp.float32)

    def _ones(i, carry):
        ones_v[pl.ds(i * 16, 16)] = ovec
        return carry

    lax.fori_loop(0, _ROW_W // 16, _ones, 0)

    in1.wait()
    in2.wait()

    for q in range(_BPT // _CH):
        pltpu.make_async_copy(
            stage_a, hist_sh.at[pl.ds(sid * _BPT, _CH)], sem_r).wait()

    plsc.subcore_barrier()

    def _compute_row(j):
        for k in range(_ROW_W // 16):
            off = j * _ROW_W + k * 16
            d = ei_v[1, pl.ds(off, 16)]
            t = typ_v[pl.ds(off, 16)]
            slot = lax.shift_right_logical(d * 13108, 25)
            idx_v[j, pl.ds(k * 16, 16)] = (
                d * 128 + t - slot * (128 * _NF - _R))

    def _fire(j):
        pltpu.make_async_copy(ones_v, hist_sh.at[idx_v.at[j]], sem).start()

    def _wait_one():
        pltpu.make_async_copy(ones_v, hist_sh.at[idx_v.at[0]], sem).wait()

    def _body(j, carry):
        _compute_row(j)
        _fire(j)

        @pl.when(j >= _DEPTH)
        def _():
            _wait_one()

        return carry

    lax.fori_loop(0, _ROWS - 1, _body, 0)

    _compute_row(_ROWS - 1)
    pad_vec = jnp.full((16,), _NBINS, jnp.int32)

    @pl.when(wid < _EXTRA_FROM)
    def _():
        def _pad(i, carry):
            idx_v[_ROWS - 1, pl.ds(i * 16, 16)] = pad_vec
            return carry

        lax.fori_loop(0, _ROW_W // 16, _pad, 0)

    _fire(_ROWS - 1)
    for _ in range(_DEPTH + 1):
        _wait_one()

    plsc.subcore_barrier()

    def _rd(c, buf, s):
        return pltpu.make_async_copy(
            hist_sh.at[pl.ds(sid * _BPT + c * _CH, _CH)], buf, s)

    def _wr(c, buf, s):
        return pltpu.make_async_copy(
            buf, out_hbm.at[pl.ds(cid * _NBINS + sid * _BPT + c * _CH, _CH)],
            s)

    _rd(0, stage_a, sem_r).start()
    _rd(1, stage_b, sem_w).start()
    _rd(0, stage_a, sem_r).wait()
    _wr(0, stage_a, sem_r).start()
    _rd(1, stage_b, sem_w).wait()
    _wr(1, stage_b, sem_w).start()
    _wr(0, stage_a, sem_r).wait()
    _wr(1, stage_b, sem_w).wait()


_sc_count = pl.kernel(
    _sc_count_body,
    out_type=jax.ShapeDtypeStruct((_NUM_CORES * _NBINS,), jnp.float32),
    mesh=plsc.VectorSubcoreMesh(core_axis_name="c", subcore_axis_name="s"),
    scratch_types=[
        pltpu.VMEM((2, _EPT_PAD), jnp.int32),
        pltpu.VMEM((_EPT_PAD,), jnp.int32),
        pltpu.VMEM((_ROWS, _ROW_W), jnp.int32),
        pltpu.VMEM((_ROW_W,), jnp.float32),
        pltpu.VMEM((_CH,), jnp.float32),
        pltpu.VMEM((_CH,), jnp.float32),
        pltpu.VMEM_SHARED((_NBINS + 16,), jnp.float32),
        pltpu.SemaphoreType.DMA,
        pltpu.SemaphoreType.DMA,
        pltpu.SemaphoreType.DMA,
        pltpu.SemaphoreType.DMA,
    ],
)


_BLKF = 1280


def _tc_attn_body(x_ref, cnt_ref, eemb_ref, br_ref, basr_ref,
                  coefr_ref, out_ref):
    k = pl.program_id(1)
    x = x_ref[...]
    basr = basr_ref[...][:, :, 0]
    rw = jnp.dot(coefr_ref[...], basr,
                 preferred_element_type=jnp.float32)
    wp = rw[:, :_D]
    wq = rw[:, _D:2 * _D]
    wr = rw[:, 2 * _D:]
    p = lax.dot_general(x, wp, (((1,), (1,)), ((), ())),
                        preferred_element_type=jnp.float32)
    q = lax.dot_general(x, wq, (((1,), (1,)), ((), ())),
                        preferred_element_type=jnp.float32)
    dvec = jnp.sum(eemb_ref[...] * wr, axis=1) + br_ref[...][:, 0]
    lane = lax.broadcasted_iota(jnp.int32, (128, _R), 0)
    col = lax.broadcasted_iota(jnp.int32, (128, _R), 1)
    sel = (lane == col + _R * k).astype(jnp.float32)
    c3 = cnt_ref[...]
    csum = c3[0] + c3[1]
    cnt = lax.dot_general(csum, sel, (((1,), (0,)), ((), ())),
                          preferred_element_type=jnp.float32)
    mask = cnt > 0.0
    score = p + jnp.where(mask, q, 0.0) + dvec[None, :]
    score = jnp.where(mask, score, jnp.float32(-10000000000.0))
    m = jnp.max(score, axis=1, keepdims=True)
    e = jnp.exp(score - m)
    z = jnp.sum(e, axis=1, keepdims=True)
    s = jnp.sum(jnp.where(mask, e, 0.0), axis=1, keepdims=True) / z
    out_ref[...] = x * (1.0 + s)


_tc_attn = pl.pallas_call(
    _tc_attn_body,
    grid=(_NF // _BLKF, _FOLD),
    in_specs=[
        pl.BlockSpec((_BLKF, _D), lambda i, k: (k * (_NF // _BLKF) + i, 0)),
        pl.BlockSpec((2, _BLKF, 128), lambda i, k: (0, i, 0)),
        pl.BlockSpec((_R, _RD), lambda i, k: (0, 0)),
        pl.BlockSpec((_R, 1), lambda i, k: (0, 0)),
        pl.BlockSpec((_NB, _IN_S, 1), lambda i, k: (0, 0, 0)),
        pl.BlockSpec((_R, _NB), lambda i, k: (0, 0)),
    ],
    out_specs=pl.BlockSpec((_BLKF, _D), lambda i, k: (k * (_NF // _BLKF) + i, 0)),
    out_shape=jax.ShapeDtypeStruct((_N, _D), jnp.float32),
)


def kernel(node_emb, edge_embeddings, b_s, b_r, bases_s, coeff_s, bases_r,
           coeff_r, edge_index, edge_type):
    del b_s, bases_s, coeff_s
    counts = _sc_count(edge_index, edge_type)
    counts3 = counts.reshape(_NUM_CORES, _NF, 128)
    return _tc_attn(node_emb, counts3, edge_embeddings, b_r,
                    bases_r, coeff_r)


# speedup vs baseline: 217.8188x; 1.0478x over previous
import functools

import jax
import jax.numpy as jnp
from jax import lax
from jax.experimental import pallas as pl
from jax.experimental.pallas import tpu as pltpu
from jax.experimental.pallas import tpu_sc as plsc

_N = 10000
_E = 320000
_D = 128
_RD = 16
_R = 32
_NB = 16
_IN_S = 2 * _D + _RD
_FOLD = 4
_NF = 2560
_NBINS = _NF * 128

_NUM_CORES = 2
_NUM_SUBCORES = 16
_NUM_WORKERS = _NUM_CORES * _NUM_SUBCORES
_ROW_W = 128
_ROWS = 79
_EPT_PAD = _ROWS * _ROW_W
_BASE_ROWS = 78
_EXTRA_FROM = _NUM_WORKERS - 4
_BPT = _NBINS // _NUM_SUBCORES
_CH = 10240
_DEPTH = 8


def _sc_count_body(edge_index_hbm, edge_type_hbm, out_hbm,
                   ei_v, typ_v, idx_v, ones_v, stage_a, stage_b, hist_sh,
                   sem, sem2, sem_r, sem_w):
    cid = lax.axis_index("c")
    sid = lax.axis_index("s")
    wid = sid * _NUM_CORES + cid
    base = (wid * _BASE_ROWS + jnp.maximum(wid - _EXTRA_FROM, 0)) * _ROW_W

    in1 = pltpu.make_async_copy(
        edge_index_hbm.at[:, pl.ds(base, _EPT_PAD)], ei_v, sem2)
    in1.start()
    in2 = pltpu.make_async_copy(
        edge_type_hbm.at[pl.ds(base, _EPT_PAD)], typ_v, sem2)
    in2.start()

    zvec = jnp.zeros((16,), jnp.float32)

    def _zero(i, carry):
        stage_a[pl.ds(i * 16, 16)] = zvec
        return carry

    lax.fori_loop(0, _CH // 16, _zero, 0)
    for q in range(_BPT // _CH):
        pltpu.make_async_copy(
            stage_a, hist_sh.at[pl.ds(sid * _BPT + q * _CH, _CH)],
            sem_r).start()

    ovec = jnp.ones((16,), jnp.float32)

    def _ones(i, carry):
        ones_v[pl.ds(i * 16, 16)] = ovec
        return carry

    lax.fori_loop(0, _ROW_W // 16, _ones, 0)

    in1.wait()
    in2.wait()

    for q in range(_BPT // _CH):
        pltpu.make_async_copy(
            stage_a, hist_sh.at[pl.ds(sid * _BPT, _CH)], sem_r).wait()

    plsc.subcore_barrier()

    def _compute_row(j):
        for k in range(_ROW_W // 16):
            off = j * _ROW_W + k * 16
            d = ei_v[1, pl.ds(off, 16)]
            t = typ_v[pl.ds(off, 16)]
            slot = lax.shift_right_logical(d * 13108, 25)
            idx_v[j, pl.ds(k * 16, 16)] = (
                d * 128 + t - slot * (128 * _NF - _R))

    def _fire(j):
        pltpu.make_async_copy(ones_v, hist_sh.at[idx_v.at[j]], sem).start()

    def _wait_one():
        pltpu.make_async_copy(ones_v, hist_sh.at[idx_v.at[0]], sem).wait()

    def _body(j, carry):
        _compute_row(j)
        _fire(j)

        @pl.when(j >= _DEPTH)
        def _():
            _wait_one()

        return carry

    lax.fori_loop(0, _ROWS - 1, _body, 0)

    _compute_row(_ROWS - 1)
    pad_vec = jnp.full((16,), _NBINS, jnp.int32)

    @pl.when(wid < _EXTRA_FROM)
    def _():
        def _pad(i, carry):
            idx_v[_ROWS - 1, pl.ds(i * 16, 16)] = pad_vec
            return carry

        lax.fori_loop(0, _ROW_W // 16, _pad, 0)

    _fire(_ROWS - 1)
    for _ in range(_DEPTH + 1):
        _wait_one()

    plsc.subcore_barrier()

    def _rd(c, buf, s):
        return pltpu.make_async_copy(
            hist_sh.at[pl.ds(sid * _BPT + c * _CH, _CH)], buf, s)

    def _wr(c, buf, s):
        return pltpu.make_async_copy(
            buf, out_hbm.at[pl.ds(cid * _NBINS + sid * _BPT + c * _CH, _CH)],
            s)

    _rd(0, stage_a, sem_r).start()
    _rd(1, stage_b, sem_w).start()
    _rd(0, stage_a, sem_r).wait()
    _wr(0, stage_a, sem_r).start()
    _rd(1, stage_b, sem_w).wait()
    _wr(1, stage_b, sem_w).start()
    _wr(0, stage_a, sem_r).wait()
    _wr(1, stage_b, sem_w).wait()


_sc_count = pl.kernel(
    _sc_count_body,
    out_type=jax.ShapeDtypeStruct((_NUM_CORES * _NBINS,), jnp.float32),
    mesh=plsc.VectorSubcoreMesh(core_axis_name="c", subcore_axis_name="s"),
    scratch_types=[
        pltpu.VMEM((2, _EPT_PAD), jnp.int32),
        pltpu.VMEM((_EPT_PAD,), jnp.int32),
        pltpu.VMEM((_ROWS, _ROW_W), jnp.int32),
        pltpu.VMEM((_ROW_W,), jnp.float32),
        pltpu.VMEM((_CH,), jnp.float32),
        pltpu.VMEM((_CH,), jnp.float32),
        pltpu.VMEM_SHARED((_NBINS + 16,), jnp.float32),
        pltpu.SemaphoreType.DMA,
        pltpu.SemaphoreType.DMA,
        pltpu.SemaphoreType.DMA,
        pltpu.SemaphoreType.DMA,
    ],
)


_BLKF = 2560


def _tc_attn_body(x_ref, cnt_ref, eemb_ref, br_ref, basr_ref,
                  coefr_ref, out_ref):
    k = pl.program_id(1)
    x = x_ref[...]
    basr = basr_ref[...][:, :, 0]
    rw = jnp.dot(coefr_ref[...], basr,
                 preferred_element_type=jnp.float32)
    wp = rw[:, :_D]
    wq = rw[:, _D:2 * _D]
    wr = rw[:, 2 * _D:]
    p = lax.dot_general(x, wp, (((1,), (1,)), ((), ())),
                        preferred_element_type=jnp.float32)
    q = lax.dot_general(x, wq, (((1,), (1,)), ((), ())),
                        preferred_element_type=jnp.float32)
    dvec = jnp.sum(eemb_ref[...] * wr, axis=1) + br_ref[...][:, 0]
    lane = lax.broadcasted_iota(jnp.int32, (128, _R), 0)
    col = lax.broadcasted_iota(jnp.int32, (128, _R), 1)
    sel = (lane == col + _R * k).astype(jnp.float32)
    c3 = cnt_ref[...]
    csum = c3[0] + c3[1]
    cnt = lax.dot_general(csum, sel, (((1,), (0,)), ((), ())),
                          preferred_element_type=jnp.float32)
    mask = cnt > 0.0
    score = p + jnp.where(mask, q, 0.0) + dvec[None, :]
    score = jnp.where(mask, score, jnp.float32(-10000000000.0))
    m = jnp.max(score, axis=1, keepdims=True)
    e = jnp.exp(score - m)
    z = jnp.sum(e, axis=1, keepdims=True)
    s = jnp.sum(jnp.where(mask, e, 0.0), axis=1, keepdims=True) / z
    out_ref[...] = x * (1.0 + s)


_tc_attn = pl.pallas_call(
    _tc_attn_body,
    grid=(_NF // _BLKF, _FOLD),
    in_specs=[
        pl.BlockSpec((_BLKF, _D), lambda i, k: (k * (_NF // _BLKF) + i, 0)),
        pl.BlockSpec((2, _BLKF, 128), lambda i, k: (0, i, 0)),
        pl.BlockSpec((_R, _RD), lambda i, k: (0, 0)),
        pl.BlockSpec((_R, 1), lambda i, k: (0, 0)),
        pl.BlockSpec((_NB, _IN_S, 1), lambda i, k: (0, 0, 0)),
        pl.BlockSpec((_R, _NB), lambda i, k: (0, 0)),
    ],
    out_specs=pl.BlockSpec((_BLKF, _D), lambda i, k: (k * (_NF // _BLKF) + i, 0)),
    out_shape=jax.ShapeDtypeStruct((_N, _D), jnp.float32),
)


def kernel(node_emb, edge_embeddings, b_s, b_r, bases_s, coeff_s, bases_r,
           coeff_r, edge_index, edge_type):
    del b_s, bases_s, coeff_s
    counts = _sc_count(edge_index, edge_type)
    counts3 = counts.reshape(_NUM_CORES, _NF, 128)
    return _tc_attn(node_emb, counts3, edge_embeddings, b_r,
                    bases_r, coeff_r)
